# Initial kernel scaffold; baseline (speedup 1.0000x reference)
import jax
import jax.numpy as jnp
from jax.experimental import pallas as pl


def kernel(pred, succ, neg, weight):
    raise NotImplementedError("write your pallas kernel here")


# speedup vs baseline: 1.1736x; 1.1736x over previous
import functools

import jax
import jax.numpy as jnp
from jax.experimental import pallas as pl
from jax.experimental.pallas import tpu as pltpu
from jax.experimental.pallas import tpu_sc as plsc


def _sc_gather(table, idx2d, n, d):
    w = 128
    mesh = plsc.VectorSubcoreMesh(core_axis_name="c", subcore_axis_name="s")

    @functools.partial(
        pl.kernel,
        out_type=jax.ShapeDtypeStruct((n, d), jnp.float32),
        mesh=mesh,
        compiler_params=pltpu.CompilerParams(use_tc_tiling_on_sc=False),
    )
    def gather_kernel(table_hbm, idx_hbm, out_hbm):
        def body(idx_vmem, out_vmem):
            pltpu.sync_copy(table_hbm.at[idx_vmem.at[0]], out_vmem)

        pltpu.emit_pipeline(
            body,
            grid=(n // w,),
            in_specs=[pl.BlockSpec((1, w), lambda i: (0, i))],
            out_specs=[pl.BlockSpec((w, d), lambda i: (i, 0))],
            core_axis_name=("c", "s"),
            dimension_semantics=(pltpu.PARALLEL,),
        )(idx_hbm, out_hbm)

    return gather_kernel(table, idx2d)


def _tc_loss(g, bb):
    s, b, dim = g.shape
    nsteps = b // bb
    inv_b = 1.0 / b

    def body(g_ref, out_ref):
        i = pl.program_id(0)
        blk = g_ref[...]
        u = blk[0]
        v = blk[1:]
        eps = 1e-7
        un = jnp.sum(u * u, axis=-1)
        vn = jnp.sum(v * v, axis=-1)
        diff = v - u[None]
        sq = jnp.sum(diff * diff, axis=-1)
        denom = jnp.maximum((1.0 - un)[None, :] * (1.0 - vn), eps)
        arg = jnp.maximum(1.0 + 2.0 * sq / denom, 1.0 + eps)
        dist = -jnp.log(arg + jnp.sqrt((arg - 1.0) * (arg + 1.0)))
        m = jnp.max(dist, axis=0)
        lz = jnp.log(jnp.sum(jnp.exp(dist - m[None, :]), axis=0)) + m
        part = jnp.sum(lz - dist[0])

        @pl.when(i == 0)
        def _():
            out_ref[...] = jnp.zeros_like(out_ref)

        out_ref[...] = out_ref[...] + part

        @pl.when(i == nsteps - 1)
        def _():
            out_ref[...] = out_ref[...] * inv_b

    return pl.pallas_call(
        body,
        grid=(nsteps,),
        in_specs=[pl.BlockSpec((s, bb, dim), lambda i: (0, i, 0))],
        out_specs=pl.BlockSpec((1, 1), lambda i: (0, 0)),
        out_shape=jax.ShapeDtypeStruct((1, 1), jnp.float32),
    )(g)


def kernel(pred, succ, neg, weight):
    b = pred.shape[0]
    nneg = neg.shape[1]
    dim = weight.shape[1]
    s = 2 + nneg

    idx = jnp.concatenate(
        [succ[None, :], pred[None, :], neg.T], axis=0
    ).astype(jnp.int32)
    n = s * b
    gathered = _sc_gather(weight, idx.reshape(1, n), n, dim)
    loss = _tc_loss(gathered.reshape(s, b, dim), bb=512)
    return loss.reshape(())


# speedup vs baseline: 1.9735x; 1.6816x over previous
import functools

import jax
import jax.numpy as jnp
from jax.experimental import pallas as pl
from jax.experimental.pallas import tpu as pltpu
from jax.experimental.pallas import tpu_sc as plsc


def _sc_gather(table, idx2d, n, d):
    w = 128
    mesh = plsc.VectorSubcoreMesh(core_axis_name="c", subcore_axis_name="s")

    @functools.partial(
        pl.kernel,
        out_type=jax.ShapeDtypeStruct((n, d), jnp.float32),
        mesh=mesh,
        compiler_params=pltpu.CompilerParams(use_tc_tiling_on_sc=False),
    )
    def gather_kernel(table_hbm, idx_hbm, out_hbm):
        def body(idx_vmem, out_vmem):
            pltpu.sync_copy(table_hbm.at[idx_vmem.at[0]], out_vmem)

        pltpu.emit_pipeline(
            body,
            grid=(n // w,),
            in_specs=[pl.BlockSpec((1, w), lambda i: (0, i))],
            out_specs=[pl.BlockSpec((w, d), lambda i: (i, 0))],
            core_axis_name=("c", "s"),
            dimension_semantics=(pltpu.PARALLEL,),
        )(idx_hbm, out_hbm)

    return gather_kernel(table, idx2d)


def _tc_loss(g2, a_mat, t_mat, s, dim, bb):
    b, sd = g2.shape
    nsteps = b // bb
    inv_b = 1.0 / b
    dn = (((1,), (0,)), ((), ()))

    def body(g_ref, a_ref, t_ref, out_ref):
        i = pl.program_id(0)
        g = g_ref[...]
        a = a_ref[...]
        t = t_ref[...]
        u = g[:, 0:dim]
        util = jax.lax.dot_general(u, t, dn, preferred_element_type=jnp.float32)
        diff = g - util
        sq_all = jax.lax.dot_general(
            diff * diff, a, dn, preferred_element_type=jnp.float32)
        ns_all = jax.lax.dot_general(
            g * g, a, dn, preferred_element_type=jnp.float32)
        un = ns_all[:, 0:1]
        vn = ns_all[:, 1:s]
        sq = sq_all[:, 1:s]
        eps = 1e-7
        denom = jnp.maximum((1.0 - un) * (1.0 - vn), eps)
        arg = jnp.maximum(1.0 + 2.0 * sq / denom, 1.0 + eps)
        dist = -jnp.log(arg + jnp.sqrt((arg - 1.0) * (arg + 1.0)))
        m = jnp.max(dist, axis=1, keepdims=True)
        lz = jnp.log(jnp.sum(jnp.exp(dist - m), axis=1, keepdims=True)) + m
        part = jnp.sum(lz - dist[:, 0:1])

        @pl.when(i == 0)
        def _():
            out_ref[...] = jnp.zeros_like(out_ref)

        out_ref[...] = out_ref[...] + part

        @pl.when(i == nsteps - 1)
        def _():
            out_ref[...] = out_ref[...] * inv_b

    return pl.pallas_call(
        body,
        grid=(nsteps,),
        in_specs=[
            pl.BlockSpec((bb, sd), lambda i: (i, 0)),
            pl.BlockSpec((sd, s), lambda i: (0, 0)),
            pl.BlockSpec((dim, sd), lambda i: (0, 0)),
        ],
        out_specs=pl.BlockSpec((1, 1), lambda i: (0, 0)),
        out_shape=jax.ShapeDtypeStruct((1, 1), jnp.float32),
    )(g2, a_mat, t_mat)


def kernel(pred, succ, neg, weight):
    b = pred.shape[0]
    nneg = neg.shape[1]
    dim = weight.shape[1]
    s = 2 + nneg
    sd = s * dim

    idx = jnp.concatenate(
        [succ[:, None], pred[:, None], neg], axis=1
    ).astype(jnp.int32)
    n = s * b
    gathered = _sc_gather(weight, idx.reshape(1, n), n, dim)

    cols = jax.lax.iota(jnp.int32, sd)
    a_mat = (cols[:, None] // dim == jax.lax.iota(jnp.int32, s)[None, :])
    a_mat = a_mat.astype(jnp.float32)
    t_mat = (jax.lax.iota(jnp.int32, dim)[:, None] == cols[None, :] % dim)
    t_mat = t_mat.astype(jnp.float32)

    loss = _tc_loss(gathered.reshape(b, sd), a_mat, t_mat, s, dim, bb=1024)
    return loss.reshape(())
